# Initial kernel scaffold; baseline (speedup 1.0000x reference)
import jax
import jax.numpy as jnp
from jax.experimental import pallas as pl


def kernel(input_ids, embedding_matrix):
    raise NotImplementedError("write your pallas kernel here")


# speedup vs baseline: 1.4107x; 1.4107x over previous
import functools

import jax
import jax.numpy as jnp
from jax import lax
from jax.experimental import pallas as pl
from jax.experimental.pallas import tpu as pltpu
from jax.experimental.pallas import tpu_sc as plsc

_DIM = 32
_PAD = 0


def _make_sc_lookup(n, dim):
    info = plsc.get_sparse_core_info()
    nc, ns, lanes = info.num_cores, info.num_subcores, info.num_lanes
    nw = nc * ns
    assert n % nw == 0
    per_w = n // nw
    chunk = 3200
    assert per_w % chunk == 0
    n_chunks = per_w // chunk
    groups = chunk // lanes

    @functools.partial(
        pl.kernel,
        mesh=plsc.VectorSubcoreMesh(core_axis_name="c", subcore_axis_name="s"),
        out_type=jax.ShapeDtypeStruct((n, dim), jnp.float32),
        compiler_params=pltpu.CompilerParams(
            use_tc_tiling_on_sc=False, needs_layout_passes=False
        ),
        scratch_types=[
            pltpu.VMEM((chunk,), jnp.int32),
            pltpu.VMEM((chunk, dim), jnp.float32),
            pltpu.VMEM((lanes, dim), jnp.float32),
            pltpu.SemaphoreType.DMA,
            pltpu.SemaphoreType.DMA,
        ],
    )
    def run(table_hbm, ids_hbm, out_hbm, idx_v, rows_v, zeros_v, gsem, zsem):
        wid = lax.axis_index("s") * nc + lax.axis_index("c")
        base = wid * per_w

        for r in range(lanes):
            for h in range(dim // lanes):
                zeros_v[r, pl.ds(h * lanes, lanes)] = jnp.zeros(
                    (lanes,), jnp.float32
                )

        for c in range(n_chunks):
            cbase = base + c * chunk
            pltpu.sync_copy(ids_hbm.at[pl.ds(cbase, chunk)], idx_v)
            pltpu.async_copy(table_hbm.at[idx_v], rows_v, gsem).wait()
            pltpu.sync_copy(rows_v, out_hbm.at[pl.ds(cbase, chunk)])

            def group_body(g, _):
                v = idx_v[pl.ds(g * lanes, lanes)]
                m = v == _PAD
                npad = jnp.sum(jnp.where(m, 1, 0))

                @pl.when(npad > 0)
                def _():
                    rowvec = (
                        cbase
                        + g * lanes
                        + lax.iota(jnp.int32, lanes)
                    )
                    first = jnp.min(
                        jnp.where(m, rowvec, jnp.int32(2**30))
                    )
                    pos = jnp.where(m, rowvec, first)
                    pltpu.async_copy(zeros_v, out_hbm.at[pos], zsem).wait()

                return 0

            lax.fori_loop(0, groups, group_body, 0)

    return run


def kernel(input_ids, embedding_matrix):
    b, s = input_ids.shape
    n = b * s
    ids_flat = input_ids.reshape(n).astype(jnp.int32)
    run = _make_sc_lookup(n, _DIM)
    out = run(embedding_matrix, ids_flat)
    return out.reshape(b, s, _DIM)


# speedup vs baseline: 1.4976x; 1.0616x over previous
import functools

import jax
import jax.numpy as jnp
from jax import lax
from jax.experimental import pallas as pl
from jax.experimental.pallas import tpu as pltpu
from jax.experimental.pallas import tpu_sc as plsc

_DIM = 32
_PAD = 0
_NBUF = 4
_GLOOK = 2
_ILOOK = 3


def _make_sc_lookup(n, dim):
    info = plsc.get_sparse_core_info()
    nc, ns, lanes = info.num_cores, info.num_subcores, info.num_lanes
    nw = nc * ns
    assert n % nw == 0
    per_w = n // nw
    chunk = 800
    assert per_w % chunk == 0
    n_chunks = per_w // chunk
    groups = chunk // lanes

    @functools.partial(
        pl.kernel,
        mesh=plsc.VectorSubcoreMesh(core_axis_name="c", subcore_axis_name="s"),
        out_type=jax.ShapeDtypeStruct((n, dim), jnp.float32),
        compiler_params=pltpu.CompilerParams(
            use_tc_tiling_on_sc=False, needs_layout_passes=False
        ),
        scratch_types=[
            pltpu.VMEM((_NBUF, chunk), jnp.int32),
            pltpu.VMEM((_NBUF, chunk, dim), jnp.float32),
            [pltpu.SemaphoreType.DMA] * _NBUF,
            [pltpu.SemaphoreType.DMA] * _NBUF,
            [pltpu.SemaphoreType.DMA] * _NBUF,
        ],
    )
    def run(table_hbm, ids_hbm, out_hbm, idx_v, rows_v, isems, gsems, osems):
        wid = lax.axis_index("s") * nc + lax.axis_index("c")
        base = wid * per_w

        icopies = {}
        gcopies = {}
        ocopies = {}

        def issue_idx(c):
            b = c % _NBUF
            icopies[c] = pltpu.async_copy(
                ids_hbm.at[pl.ds(base + c * chunk, chunk)],
                idx_v.at[b],
                isems[b],
            )

        def issue_gather(c):
            b = c % _NBUF
            icopies.pop(c).wait()
            if c - _NBUF in ocopies:
                ocopies.pop(c - _NBUF).wait()
            gcopies[c] = pltpu.async_copy(
                table_hbm.at[idx_v.at[b]], rows_v.at[b], gsems[b]
            )

        def fix_pads(c):
            b = c % _NBUF
            acc = jnp.full((lanes,), 1, jnp.int32)

            def min_body(g, a):
                return jnp.minimum(a, idx_v[b, pl.ds(g * lanes, lanes)])

            acc = lax.fori_loop(0, groups, min_body, acc)

            @pl.when(jnp.min(acc) == _PAD)
            def _():
                def scan_body(g, _):
                    v = idx_v[b, pl.ds(g * lanes, lanes)]
                    m = v == _PAD
                    npad = jnp.sum(jnp.where(m, 1, 0))

                    @pl.when(npad > 0)
                    def _():
                        rvec = g * lanes + lax.iota(jnp.int32, lanes)
                        zero = jnp.zeros((lanes,), jnp.float32)
                        for j in range(dim):
                            cvec = jnp.full((lanes,), j, jnp.int32)
                            plsc.store_scatter(
                                rows_v.at[b], [rvec, cvec], zero, mask=m
                            )

                    return 0

                lax.fori_loop(0, groups, scan_body, 0)

        def issue_out(c):
            b = c % _NBUF
            gcopies.pop(c).wait()
            fix_pads(c)
            ocopies[c] = pltpu.async_copy(
                rows_v.at[b],
                out_hbm.at[pl.ds(base + c * chunk, chunk)],
                osems[b],
            )

        for c in range(min(_ILOOK, n_chunks)):
            issue_idx(c)
        for c in range(min(_GLOOK, n_chunks)):
            issue_gather(c)

        for c in range(n_chunks):
            issue_out(c)
            if c + _ILOOK < n_chunks:
                issue_idx(c + _ILOOK)
            if c + _GLOOK < n_chunks:
                issue_gather(c + _GLOOK)

        for c in sorted(ocopies):
            ocopies.pop(c).wait()

    return run


def kernel(input_ids, embedding_matrix):
    b, s = input_ids.shape
    n = b * s
    ids_flat = input_ids.reshape(n).astype(jnp.int32)
    run = _make_sc_lookup(n, _DIM)
    out = run(embedding_matrix, ids_flat)
    return out.reshape(b, s, _DIM)
